# baseline (speedup 1.0000x reference)
import functools

import jax
import jax.numpy as jnp
from jax import lax
from jax.experimental import pallas as pl
from jax.experimental.pallas import tpu as pltpu
from jax.experimental.pallas import tpu_sc as plsc


DIM = 1024
LANES = 16

_NUM_WORKERS = 32
_CHUNK_ROWS = 8
_XSLOTS = 3


def _sc_body(x_hbm, emb_hbm, out_hbm, *scratch, batch, rows_per_worker):
    xslots = [list(scratch[s * batch:(s + 1) * batch]) for s in range(_XSLOTS)]
    rest = scratch[_XSLOTS * batch:]
    ebuf = rest[0:2]
    esem = rest[2:4]
    lsem = rest[4:4 + _XSLOTS]
    ssem = rest[4 + _XSLOTS:4 + 2 * _XSLOTS]

    wid = lax.axis_index("s") * 2 + lax.axis_index("c")
    row0 = wid * rows_per_worker
    nc = rows_per_worker // _CHUNK_ROWS

    def e_src(c):
        return emb_hbm.at[pl.ds(row0 + c * _CHUNK_ROWS, _CHUNK_ROWS), :]

    def x_src(c, b):
        return x_hbm.at[b, pl.ds(row0 + c * _CHUNK_ROWS, _CHUNK_ROWS), :]

    def o_dst(c, b):
        return out_hbm.at[b, pl.ds(row0 + c * _CHUNK_ROWS, _CHUNK_ROWS), :]

    def do_chunk(c, kx, ke, store_guard):
        kn = (kx + 1) % _XSLOTS
        pltpu.make_async_copy(e_src(c), ebuf[ke], esem[ke]).wait()

        def issue_next_emb():
            pltpu.make_async_copy(e_src(c + 1), ebuf[1 - ke], esem[1 - ke]).start()

        if isinstance(c, int) and c + 1 >= nc:
            pass
        else:
            issue_next_emb()

        for b in range(batch):
            pltpu.make_async_copy(x_src(c, b), xslots[kx][b], lsem[kx]).wait()

        def drain_and_load():
            for b in range(batch):
                pltpu.make_async_copy(xslots[kn][b], o_dst(c - 2, b), ssem[kn]).wait()
            for b in range(batch):
                pltpu.make_async_copy(x_src(c + 1, b), xslots[kn][b], lsem[kn]).start()

        def load_only():
            for b in range(batch):
                pltpu.make_async_copy(x_src(c + 1, b), xslots[kn][b], lsem[kn]).start()

        last = isinstance(c, int) and c + 1 >= nc
        if not last:
            if store_guard is None:
                drain_and_load()
            else:
                @pl.when(store_guard)
                def _():
                    drain_and_load()

                @pl.when(jnp.logical_not(store_guard))
                def _():
                    load_only()

        bufs = tuple(xslots[kx])

        def row_add(r, _, bufs=bufs, ke=ke):
            for u in range(DIM // LANES):
                sl = pl.ds(u * LANES, LANES)
                ev = ebuf[ke][r, sl]
                for xb in bufs:
                    xb[r, sl] = xb[r, sl] + ev
            return 0

        lax.fori_loop(0, _CHUNK_ROWS, row_add, 0, unroll=False)

        for b in range(batch):
            pltpu.make_async_copy(xslots[kx][b], o_dst(c, b), ssem[kx]).start()

    pltpu.make_async_copy(e_src(0), ebuf[0], esem[0]).start()
    for b in range(batch):
        pltpu.make_async_copy(x_src(0, b), xslots[0][b], lsem[0]).start()

    ngroups = nc // 6
    def group(c6, _):
        for k in range(6):
            c = c6 * 6 + k
            guard = (c6 > 0) if k < 2 else None
            do_chunk(c, k % _XSLOTS, k % 2, guard)
        return 0

    lax.fori_loop(0, ngroups, group, 0, unroll=False)
    for c in range(ngroups * 6, nc):
        do_chunk(c, c % _XSLOTS, c % 2, None)

    for c in (nc - 3, nc - 2, nc - 1):
        for b in range(batch):
            pltpu.make_async_copy(xslots[c % _XSLOTS][b], o_dst(c, b),
                                  ssem[c % _XSLOTS]).wait()


def kernel(x, emb):
    batch, seq, dim = x.shape
    rows_per_worker = seq // _NUM_WORKERS
    mesh = plsc.VectorSubcoreMesh(core_axis_name="c", subcore_axis_name="s")
    body = functools.partial(
        _sc_body, batch=batch, rows_per_worker=rows_per_worker
    )
    vmem = pltpu.VMEM((_CHUNK_ROWS, DIM), jnp.float32)
    return pl.kernel(
        body,
        out_type=jax.ShapeDtypeStruct((batch, seq, dim), jnp.float32),
        mesh=mesh,
        scratch_types=(
            [vmem] * (_XSLOTS * batch)
            + [vmem] * 2
            + [pltpu.SemaphoreType.DMA] * 2
            + [pltpu.SemaphoreType.DMA] * _XSLOTS
            + [pltpu.SemaphoreType.DMA] * _XSLOTS
        ),
    )(x, emb[:seq])


# speedup vs baseline: 1.2474x; 1.2474x over previous
import functools

import jax
import jax.numpy as jnp
from jax import lax
from jax.experimental import pallas as pl
from jax.experimental.pallas import tpu as pltpu
from jax.experimental.pallas import tpu_sc as plsc


DIM = 1024
LANES = 16

_NUM_WORKERS = 32
_CHUNK_ROWS = 8


def _sc_body(x_hbm, emb_hbm, out_hbm, *scratch, batch, rows_per_worker):
    npairs = batch // 2
    xbuf = [list(scratch[h * batch:(h + 1) * batch]) for h in range(2)]
    rest = scratch[2 * batch:]
    obuf = [list(rest[pr * 2:(pr + 1) * 2]) for pr in range(npairs)]
    rest = rest[npairs * 2:]
    ebuf = rest[0:2]
    esem = rest[2:4]
    lsem = rest[4:6]
    ssem = rest[6:8]

    wid = lax.axis_index("s") * 2 + lax.axis_index("c")
    row0 = wid * rows_per_worker
    nc = rows_per_worker // _CHUNK_ROWS

    def e_src(c):
        return emb_hbm.at[pl.ds(row0 + c * _CHUNK_ROWS, _CHUNK_ROWS), :]

    def x_src(c, b):
        return x_hbm.at[b, pl.ds(row0 + c * _CHUNK_ROWS, _CHUNK_ROWS), :]

    def o_dst(c, b):
        return out_hbm.at[b, pl.ds(row0 + c * _CHUNK_ROWS, _CHUNK_ROWS), :]

    def do_chunk(c, h, eguard, lguard, sguard):
        def when(g, fn):
            if g is None:
                fn()
            else:
                pl.when(g)(fn)

        pltpu.make_async_copy(e_src(c), ebuf[h], esem[h]).wait()
        when(eguard, lambda: pltpu.make_async_copy(
            e_src(c + 1), ebuf[1 - h], esem[1 - h]).start())

        def start_loads():
            for b in range(batch):
                pltpu.make_async_copy(x_src(c + 1, b), xbuf[1 - h][b],
                                      lsem[1 - h]).start()
        when(lguard, start_loads)

        for pr in range(npairs):
            b0, b1 = 2 * pr, 2 * pr + 1
            pltpu.make_async_copy(x_src(c, b0), xbuf[h][b0], lsem[h]).wait()
            pltpu.make_async_copy(x_src(c, b1), xbuf[h][b1], lsem[h]).wait()

            def drain(pr=pr, b0=b0, b1=b1):
                pltpu.make_async_copy(obuf[pr][0], o_dst(c - 1, b0),
                                      ssem[pr]).wait()
                pltpu.make_async_copy(obuf[pr][1], o_dst(c - 1, b1),
                                      ssem[pr]).wait()
            when(sguard, drain)

            o0, o1 = obuf[pr][0], obuf[pr][1]
            x0, x1 = xbuf[h][b0], xbuf[h][b1]

            @functools.partial(plsc.parallel_loop, 0, _CHUNK_ROWS)
            def _(r, o0=o0, o1=o1, x0=x0, x1=x1, h=h):
                for u in range(DIM // LANES):
                    sl = pl.ds(u * LANES, LANES)
                    ev = ebuf[h][r, sl]
                    o0[r, sl] = x0[r, sl] + ev
                    o1[r, sl] = x1[r, sl] + ev

            pltpu.make_async_copy(obuf[pr][0], o_dst(c, b0), ssem[pr]).start()
            pltpu.make_async_copy(obuf[pr][1], o_dst(c, b1), ssem[pr]).start()

    pltpu.make_async_copy(e_src(0), ebuf[0], esem[0]).start()
    for b in range(batch):
        pltpu.make_async_copy(x_src(0, b), xbuf[0][b], lsem[0]).start()

    def pair_of_chunks(c2, _):
        c = c2 * 2
        do_chunk(c, 0, None, None, c2 > 0)
        do_chunk(c + 1, 1, c2 < nc // 2 - 1, c2 < nc // 2 - 1, None)
        return 0

    lax.fori_loop(0, nc // 2, pair_of_chunks, 0, unroll=False)

    for pr in range(npairs):
        pltpu.make_async_copy(obuf[pr][0], o_dst(nc - 1, 2 * pr),
                              ssem[pr]).wait()
        pltpu.make_async_copy(obuf[pr][1], o_dst(nc - 1, 2 * pr + 1),
                              ssem[pr]).wait()


def kernel(x, emb):
    batch, seq, dim = x.shape
    rows_per_worker = seq // _NUM_WORKERS
    mesh = plsc.VectorSubcoreMesh(core_axis_name="c", subcore_axis_name="s")
    body = functools.partial(
        _sc_body, batch=batch, rows_per_worker=rows_per_worker
    )
    vmem = pltpu.VMEM((_CHUNK_ROWS, DIM), jnp.float32)
    return pl.kernel(
        body,
        out_type=jax.ShapeDtypeStruct((batch, seq, dim), jnp.float32),
        mesh=mesh,
        scratch_types=(
            [vmem] * (2 * batch)
            + [vmem] * batch
            + [vmem] * 2
            + [pltpu.SemaphoreType.DMA] * 2
            + [pltpu.SemaphoreType.DMA] * 2
            + [pltpu.SemaphoreType.DMA] * 2
        ),
    )(x, emb[:seq])
